# Initial kernel scaffold; baseline (speedup 1.0000x reference)
import jax
import jax.numpy as jnp
from jax.experimental import pallas as pl


def kernel(x, edge_index, Ws1, Wn1, b1, g1, be1, Ws2, Wn2, b2, g2, be2, Ws3, Wn3, b3):
    raise NotImplementedError("write your pallas kernel here")


# speedup vs baseline: 1.9238x; 1.9238x over previous
import functools

import jax
import jax.numpy as jnp
from jax import lax
from jax.experimental import pallas as pl
from jax.experimental.pallas import tpu as pltpu
from jax.experimental.pallas import tpu_sc as plsc

N = 10000
E = 320000
D_IN = 128
D_H = 128
D_OUT = 40

N_PAD = 10240
RPT = N_PAD // 16
CHUNK = 64
NTILE = 32
NBLOCK = 4
IB = 40
NCH = NBLOCK * IB
E_PAD = NTILE * NCH * CHUNK

BLK = 512
GRID = N_PAD // BLK


def _sc_scatter_fn(D):
    mesh = plsc.VectorSubcoreMesh(core_axis_name="c", subcore_axis_name="s")

    @functools.partial(
        pl.kernel,
        out_type=jax.ShapeDtypeStruct((2, N_PAD, D), jnp.float32),
        mesh=mesh,
        compiler_params=pltpu.CompilerParams(use_tc_tiling_on_sc=False),
        scratch_types=[
            pltpu.VMEM((IB + 2, CHUNK), jnp.int32),
            pltpu.VMEM((IB, CHUNK), jnp.int32),
            pltpu.VMEM((CHUNK, D), jnp.float32),
            pltpu.VMEM((CHUNK, D), jnp.float32),
            pltpu.VMEM_SHARED((N_PAD, D), jnp.float32),
            pltpu.SemaphoreType.DMA,
            pltpu.SemaphoreType.DMA,
        ],
    )
    def sc_scatter(y_hbm, src_hbm, dst_hbm, zero_hbm, out_hbm,
                   src_v, dst_v, rows0, rows1, acc_sh, sem0, sem1):
        c = lax.axis_index("c")
        s = lax.axis_index("s")
        tile = c * 16 + s
        pltpu.sync_copy(zero_hbm, acc_sh.at[pl.ds(s * RPT, RPT)])
        plsc.subcore_barrier()

        @pl.loop(0, NBLOCK)
        def _(bi):
            pltpu.sync_copy(src_hbm.at[tile, bi], src_v)
            pltpu.sync_copy(dst_hbm.at[tile, bi], dst_v)
            pltpu.async_copy(y_hbm.at[src_v.at[0]], rows0, sem0)
            pltpu.async_copy(y_hbm.at[src_v.at[1]], rows1, sem1)

            @pl.loop(0, IB, step=2)
            def _(j):
                pltpu.make_async_copy(y_hbm.at[src_v.at[0]], rows0, sem0).wait()
                pltpu.sync_copy(rows0, acc_sh.at[dst_v.at[j]], add=True)
                pltpu.async_copy(y_hbm.at[src_v.at[j + 2]], rows0, sem0)
                pltpu.make_async_copy(y_hbm.at[src_v.at[1]], rows1, sem1).wait()
                pltpu.sync_copy(rows1, acc_sh.at[dst_v.at[j + 1]], add=True)
                pltpu.async_copy(y_hbm.at[src_v.at[j + 3]], rows1, sem1)

            pltpu.make_async_copy(y_hbm.at[src_v.at[0]], rows0, sem0).wait()
            pltpu.make_async_copy(y_hbm.at[src_v.at[1]], rows1, sem1).wait()

        plsc.subcore_barrier()
        pltpu.sync_copy(acc_sh.at[pl.ds(s * RPT, RPT)],
                        out_hbm.at[c, pl.ds(s * RPT, RPT)])

    return sc_scatter


def _segment_sum(y, src4, dst4, D):
    zero = jnp.zeros((RPT, D), jnp.float32)
    return _sc_scatter_fn(D)(y, src4, dst4, zero)


def _mm0_body(x_ref, wn_ref, out_ref):
    blk = x_ref[...]
    y = jnp.dot(blk, wn_ref[...], preferred_element_type=jnp.float32)
    ones = jnp.ones((blk.shape[0], 1), jnp.float32)
    zpad = jnp.zeros((blk.shape[0], 15), jnp.float32)
    out_ref[...] = jnp.concatenate([y, ones, zpad], axis=1)


def _mm0(x_pad, Wn1):
    return pl.pallas_call(
        _mm0_body,
        grid=(GRID,),
        in_specs=[
            pl.BlockSpec((BLK, D_IN), lambda i: (i, 0)),
            pl.BlockSpec((D_IN, D_H), lambda i: (0, 0)),
        ],
        out_specs=pl.BlockSpec((BLK, 144), lambda i: (i, 0)),
        out_shape=jax.ShapeDtypeStruct((N_PAD, 144), jnp.float32),
    )(x_pad, Wn1)


def _pre1_body(x_ref, s0_ref, s1_ref, ws_ref, b_ref, h_ref, rdeg_ref, stats_ref):
    i = pl.program_id(0)
    ssum = s0_ref[0] + s1_ref[0]
    deg = ssum[:, 128:129]
    rdeg = 1.0 / jnp.maximum(deg, 1.0)
    h = (jnp.dot(x_ref[...], ws_ref[...], preferred_element_type=jnp.float32)
         + ssum[:, :128] * rdeg + b_ref[...])
    h_ref[...] = h
    rdeg_ref[...] = rdeg
    rows = i * BLK + lax.broadcasted_iota(jnp.int32, (BLK, 1), 0)
    hm = jnp.where(rows < N, h, 0.0)
    st = jnp.stack([jnp.sum(hm, axis=0), jnp.sum(hm * hm, axis=0)])

    @pl.when(i == 0)
    def _():
        stats_ref[...] = st

    @pl.when(i > 0)
    def _():
        stats_ref[...] += st


def _pre1(x_pad, S, Ws1, b1):
    return pl.pallas_call(
        _pre1_body,
        grid=(GRID,),
        in_specs=[
            pl.BlockSpec((BLK, D_IN), lambda i: (i, 0)),
            pl.BlockSpec((1, BLK, 144), lambda i: (0, i, 0)),
            pl.BlockSpec((1, BLK, 144), lambda i: (1, i, 0)),
            pl.BlockSpec((D_IN, D_H), lambda i: (0, 0)),
            pl.BlockSpec((1, D_H), lambda i: (0, 0)),
        ],
        out_specs=[
            pl.BlockSpec((BLK, D_H), lambda i: (i, 0)),
            pl.BlockSpec((BLK, 1), lambda i: (i, 0)),
            pl.BlockSpec((2, D_H), lambda i: (0, 0)),
        ],
        out_shape=[
            jax.ShapeDtypeStruct((N_PAD, D_H), jnp.float32),
            jax.ShapeDtypeStruct((N_PAD, 1), jnp.float32),
            jax.ShapeDtypeStruct((2, D_H), jnp.float32),
        ],
    )(x_pad, S[:, :, :], S[:, :, :], Ws1.astype(jnp.float32), b1.reshape(1, -1))


def _pre2_body(hin_ref, s0_ref, s1_ref, rdeg_ref, ws_ref, b_ref, h_ref, stats_ref):
    i = pl.program_id(0)
    ssum = s0_ref[0] + s1_ref[0]
    h = (jnp.dot(hin_ref[...], ws_ref[...], preferred_element_type=jnp.float32)
         + ssum * rdeg_ref[...] + b_ref[...])
    h_ref[...] = h
    rows = i * BLK + lax.broadcasted_iota(jnp.int32, (BLK, 1), 0)
    hm = jnp.where(rows < N, h, 0.0)
    st = jnp.stack([jnp.sum(hm, axis=0), jnp.sum(hm * hm, axis=0)])

    @pl.when(i == 0)
    def _():
        stats_ref[...] = st

    @pl.when(i > 0)
    def _():
        stats_ref[...] += st


def _pre2(h_in, S, rdeg, Ws, b):
    return pl.pallas_call(
        _pre2_body,
        grid=(GRID,),
        in_specs=[
            pl.BlockSpec((BLK, D_H), lambda i: (i, 0)),
            pl.BlockSpec((1, BLK, D_H), lambda i: (0, i, 0)),
            pl.BlockSpec((1, BLK, D_H), lambda i: (1, i, 0)),
            pl.BlockSpec((BLK, 1), lambda i: (i, 0)),
            pl.BlockSpec((D_H, D_H), lambda i: (0, 0)),
            pl.BlockSpec((1, D_H), lambda i: (0, 0)),
        ],
        out_specs=[
            pl.BlockSpec((BLK, D_H), lambda i: (i, 0)),
            pl.BlockSpec((2, D_H), lambda i: (0, 0)),
        ],
        out_shape=[
            jax.ShapeDtypeStruct((N_PAD, D_H), jnp.float32),
            jax.ShapeDtypeStruct((2, D_H), jnp.float32),
        ],
    )(h_in, S, S, rdeg, Ws, b.reshape(1, -1))


def _post_body(dn, hpre_ref, stats_ref, g_ref, be_ref, wn_ref, hact_ref, y_ref):
    mu = stats_ref[0:1, :] * (1.0 / N)
    var = stats_ref[1:2, :] * (1.0 / N) - mu * mu
    rstd = lax.rsqrt(var + 1e-5)
    h = (hpre_ref[...] - mu) * (rstd * g_ref[...]) + be_ref[...]
    h = jnp.maximum(h, 0.0)
    hact_ref[...] = h
    y = jnp.dot(h, wn_ref[...], preferred_element_type=jnp.float32)
    if dn > wn_ref.shape[1]:
        y = jnp.concatenate(
            [y, jnp.zeros((y.shape[0], dn - wn_ref.shape[1]), jnp.float32)], axis=1)
    y_ref[...] = y


def _post(hpre, stats, g, be, Wn_next, dn):
    return pl.pallas_call(
        functools.partial(_post_body, dn),
        grid=(GRID,),
        in_specs=[
            pl.BlockSpec((BLK, D_H), lambda i: (i, 0)),
            pl.BlockSpec((2, D_H), lambda i: (0, 0)),
            pl.BlockSpec((1, D_H), lambda i: (0, 0)),
            pl.BlockSpec((1, D_H), lambda i: (0, 0)),
            pl.BlockSpec(Wn_next.shape, lambda i: (0, 0)),
        ],
        out_specs=[
            pl.BlockSpec((BLK, D_H), lambda i: (i, 0)),
            pl.BlockSpec((BLK, dn), lambda i: (i, 0)),
        ],
        out_shape=[
            jax.ShapeDtypeStruct((N_PAD, D_H), jnp.float32),
            jax.ShapeDtypeStruct((N_PAD, dn), jnp.float32),
        ],
    )(hpre, stats, g.reshape(1, -1), be.reshape(1, -1), Wn_next)


def _final_body(hin_ref, s0_ref, s1_ref, rdeg_ref, ws_ref, b_ref, out_ref):
    ssum = s0_ref[0] + s1_ref[0]
    h = (jnp.dot(hin_ref[...], ws_ref[...], preferred_element_type=jnp.float32)
         + ssum[:, :D_OUT] * rdeg_ref[...] + b_ref[...])
    m = jnp.max(h, axis=1, keepdims=True)
    e = jnp.exp(h - m)
    lse = jnp.log(jnp.sum(e, axis=1, keepdims=True))
    out_ref[...] = h - m - lse


def _final(h_in, S, rdeg, Ws3, b3):
    return pl.pallas_call(
        _final_body,
        grid=(GRID,),
        in_specs=[
            pl.BlockSpec((BLK, D_H), lambda i: (i, 0)),
            pl.BlockSpec((1, BLK, 48), lambda i: (0, i, 0)),
            pl.BlockSpec((1, BLK, 48), lambda i: (1, i, 0)),
            pl.BlockSpec((BLK, 1), lambda i: (i, 0)),
            pl.BlockSpec((D_H, D_OUT), lambda i: (0, 0)),
            pl.BlockSpec((1, D_OUT), lambda i: (0, 0)),
        ],
        out_specs=pl.BlockSpec((BLK, D_OUT), lambda i: (i, 0)),
        out_shape=jax.ShapeDtypeStruct((N_PAD, D_OUT), jnp.float32),
    )(h_in, S, S, rdeg, Ws3, b3.reshape(1, -1))


def kernel(x, edge_index, Ws1, Wn1, b1, g1, be1, Ws2, Wn2, b2, g2, be2, Ws3, Wn3, b3):
    x_pad = jnp.pad(x, ((0, N_PAD - N), (0, 0)))
    src = edge_index[0]
    dst = edge_index[1]
    src_p = jnp.pad(src, (0, E_PAD - E)).reshape(NTILE, NBLOCK, IB, CHUNK)
    src4 = jnp.pad(src_p, ((0, 0), (0, 0), (0, 2), (0, 0)))
    dst4 = jnp.pad(dst, (0, E_PAD - E), constant_values=N).reshape(
        NTILE, NBLOCK, IB, CHUNK)

    y1 = _mm0(x_pad, Wn1)
    S1 = _segment_sum(y1, src4, dst4, 144)
    hpre1, rdeg, st1 = _pre1(x_pad, S1, Ws1, b1)
    h1, y2 = _post(hpre1, st1, g1, be1, Wn2, D_H)

    S2 = _segment_sum(y2, src4, dst4, D_H)
    hpre2, st2 = _pre2(h1, S2, rdeg, Ws2, b2)
    h2, y3 = _post(hpre2, st2, g2, be2, Wn3, 48)

    S3 = _segment_sum(y3, src4, dst4, 48)
    out = _final(h2, S3, rdeg, Ws3, b3)
    return out[:N, :]


# speedup vs baseline: 4.3588x; 2.2658x over previous
import functools

import jax
import jax.numpy as jnp
from jax import lax
from jax.experimental import pallas as pl
from jax.experimental.pallas import tpu as pltpu
from jax.experimental.pallas import tpu_sc as plsc

N = 10000
E = 320000
D_IN = 128
D_H = 128
D_OUT = 40
DDEG = 16

N_PAD = 10048
RPT = N_PAD // 16
CHUNK = 64
NTILE = 32
NBLOCK = 20
IB = 8
NBUF = 8
NCH = NBLOCK * IB
E_PAD = NTILE * NCH * CHUNK

BLK = 1256
GRID = N_PAD // BLK


def _sc_scatter_fn(D, with_deg):
    mesh = plsc.VectorSubcoreMesh(core_axis_name="c", subcore_axis_name="s")

    out_type = [jax.ShapeDtypeStruct((2, N_PAD, D), jnp.float32)]
    scratch = [
        pltpu.VMEM((2, IB, CHUNK), jnp.int32),
        pltpu.VMEM((2, IB, CHUNK), jnp.int32),
        [pltpu.VMEM((CHUNK, D), jnp.float32) for _ in range(NBUF)],
        pltpu.VMEM_SHARED((N_PAD, D), jnp.float32),
        [pltpu.SemaphoreType.DMA for _ in range(NBUF)],
    ]
    if with_deg:
        out_type.append(jax.ShapeDtypeStruct((2, N_PAD, DDEG), jnp.float32))
        scratch += [
            pltpu.VMEM((CHUNK, DDEG), jnp.float32),
            pltpu.VMEM_SHARED((N_PAD, DDEG), jnp.float32),
        ]

    @functools.partial(
        pl.kernel,
        out_type=out_type,
        mesh=mesh,
        compiler_params=pltpu.CompilerParams(use_tc_tiling_on_sc=False),
        scratch_types=scratch,
    )
    def sc_scatter(y_hbm, src_hbm, dst_hbm, zero_hbm, *rest):
        if with_deg:
            (ones_hbm, zdeg_hbm, out_hbm, deg_hbm,
             src_v, dst_v, rows, acc_sh, sems, ones_v, dacc_sh) = rest
        else:
            out_hbm, src_v, dst_v, rows, acc_sh, sems = rest
        c = lax.axis_index("c")
        s = lax.axis_index("s")
        tile = c * 16 + s
        pltpu.sync_copy(zero_hbm, acc_sh.at[pl.ds(s * RPT, RPT)])
        if with_deg:
            pltpu.sync_copy(zdeg_hbm, dacc_sh.at[pl.ds(s * RPT, RPT)])
            pltpu.sync_copy(ones_hbm, ones_v)
        plsc.subcore_barrier()

        def scat(b, didx):
            pltpu.sync_copy(rows[b], acc_sh.at[didx], add=True)
            if with_deg:
                pltpu.sync_copy(ones_v, dacc_sh.at[didx], add=True)

        pltpu.sync_copy(src_hbm.at[tile, 0], src_v.at[0])
        pltpu.sync_copy(dst_hbm.at[tile, 0], dst_v.at[0])
        for b in range(NBUF):
            pltpu.async_copy(y_hbm.at[src_v.at[0, b]], rows[b], sems[b])

        @pl.loop(0, NBLOCK)
        def _(bi):
            cur = lax.rem(bi, 2)
            nxt = 1 - cur

            @pl.when(bi + 1 < NBLOCK)
            def _():
                pltpu.sync_copy(src_hbm.at[tile, bi + 1], src_v.at[nxt])
                pltpu.sync_copy(dst_hbm.at[tile, bi + 1], dst_v.at[nxt])

            if IB > NBUF:
                @pl.loop(0, IB - NBUF, step=NBUF)
                def _(j):
                    for b in range(NBUF):
                        pltpu.make_async_copy(
                            y_hbm.at[src_v.at[0, 0]], rows[b], sems[b]).wait()
                        scat(b, dst_v.at[cur, j + b])
                        pltpu.async_copy(
                            y_hbm.at[src_v.at[cur, j + b + NBUF]], rows[b], sems[b])

            for b in range(NBUF):
                pltpu.make_async_copy(
                    y_hbm.at[src_v.at[0, 0]], rows[b], sems[b]).wait()
                scat(b, dst_v.at[cur, IB - NBUF + b])

                @pl.when(bi + 1 < NBLOCK)
                def _():
                    pltpu.async_copy(y_hbm.at[src_v.at[nxt, b]], rows[b], sems[b])

        plsc.subcore_barrier()
        pltpu.sync_copy(acc_sh.at[pl.ds(s * RPT, RPT)],
                        out_hbm.at[c, pl.ds(s * RPT, RPT)])
        if with_deg:
            pltpu.sync_copy(dacc_sh.at[pl.ds(s * RPT, RPT)],
                            deg_hbm.at[c, pl.ds(s * RPT, RPT)])

    return sc_scatter


def _segment_sum(y, src4, dst4, with_deg=False):
    D = y.shape[1]
    zero = jnp.zeros((RPT, D), jnp.float32)
    if with_deg:
        ones = jnp.ones((CHUNK, DDEG), jnp.float32)
        zdeg = jnp.zeros((RPT, DDEG), jnp.float32)
        out = _sc_scatter_fn(D, True)(y, src4, dst4, zero, ones, zdeg)
        return out[0], out[1]
    out = _sc_scatter_fn(D, False)(y, src4, dst4, zero)
    return out[0] if isinstance(out, (list, tuple)) else out


def _mm0_body(x_ref, wn_ref, ya_ref, yb_ref):
    y = jnp.dot(x_ref[...], wn_ref[...], preferred_element_type=jnp.float32)
    ya_ref[...] = y[:, :64]
    yb_ref[...] = y[:, 64:]


def _mm0(x_pad, Wn1):
    return pl.pallas_call(
        _mm0_body,
        grid=(GRID,),
        in_specs=[
            pl.BlockSpec((BLK, D_IN), lambda i: (i, 0)),
            pl.BlockSpec((D_IN, D_H), lambda i: (0, 0)),
        ],
        out_specs=[
            pl.BlockSpec((BLK, 64), lambda i: (i, 0)),
            pl.BlockSpec((BLK, 64), lambda i: (i, 0)),
        ],
        out_shape=[
            jax.ShapeDtypeStruct((N_PAD, 64), jnp.float32),
            jax.ShapeDtypeStruct((N_PAD, 64), jnp.float32),
        ],
    )(x_pad, Wn1)


def _stats_update(i, h, stats_ref):
    rows = i * BLK + lax.broadcasted_iota(jnp.int32, (BLK, 1), 0)
    hm = jnp.where(rows < N, h, 0.0)
    st = jnp.stack([jnp.sum(hm, axis=0), jnp.sum(hm * hm, axis=0)])

    @pl.when(i == 0)
    def _():
        stats_ref[...] = st

    @pl.when(i > 0)
    def _():
        stats_ref[...] += st


def _pre1_body(x_ref, sa0, sa1, sb0, sb1, sd0, sd1, ws_ref, b_ref,
               h_ref, rdeg_ref, stats_ref):
    i = pl.program_id(0)
    ssum = jnp.concatenate([sa0[0] + sa1[0], sb0[0] + sb1[0]], axis=1)
    deg = (sd0[0] + sd1[0])[:, 0:1]
    rdeg = 1.0 / jnp.maximum(deg, 1.0)
    h = (jnp.dot(x_ref[...], ws_ref[...], preferred_element_type=jnp.float32)
         + ssum * rdeg + b_ref[...])
    h_ref[...] = h
    rdeg_ref[...] = rdeg
    _stats_update(i, h, stats_ref)


def _plane_specs(cols):
    return [pl.BlockSpec((1, BLK, cols), lambda i: (0, i, 0)),
            pl.BlockSpec((1, BLK, cols), lambda i: (1, i, 0))]


def _pre1(x_pad, Sa, Sb, Sd, Ws1, b1):
    return pl.pallas_call(
        _pre1_body,
        grid=(GRID,),
        in_specs=[pl.BlockSpec((BLK, D_IN), lambda i: (i, 0))]
        + _plane_specs(64) + _plane_specs(64) + _plane_specs(DDEG)
        + [
            pl.BlockSpec((D_IN, D_H), lambda i: (0, 0)),
            pl.BlockSpec((1, D_H), lambda i: (0, 0)),
        ],
        out_specs=[
            pl.BlockSpec((BLK, D_H), lambda i: (i, 0)),
            pl.BlockSpec((BLK, 1), lambda i: (i, 0)),
            pl.BlockSpec((2, D_H), lambda i: (0, 0)),
        ],
        out_shape=[
            jax.ShapeDtypeStruct((N_PAD, D_H), jnp.float32),
            jax.ShapeDtypeStruct((N_PAD, 1), jnp.float32),
            jax.ShapeDtypeStruct((2, D_H), jnp.float32),
        ],
    )(x_pad, Sa, Sa, Sb, Sb, Sd, Sd, Ws1, b1.reshape(1, -1))


def _pre2_body(hin_ref, sa0, sa1, sb0, sb1, rdeg_ref, ws_ref, b_ref,
               h_ref, stats_ref):
    i = pl.program_id(0)
    ssum = jnp.concatenate([sa0[0] + sa1[0], sb0[0] + sb1[0]], axis=1)
    h = (jnp.dot(hin_ref[...], ws_ref[...], preferred_element_type=jnp.float32)
         + ssum * rdeg_ref[...] + b_ref[...])
    h_ref[...] = h
    _stats_update(i, h, stats_ref)


def _pre2(h_in, Sa, Sb, rdeg, Ws, b):
    return pl.pallas_call(
        _pre2_body,
        grid=(GRID,),
        in_specs=[pl.BlockSpec((BLK, D_H), lambda i: (i, 0))]
        + _plane_specs(64) + _plane_specs(64)
        + [
            pl.BlockSpec((BLK, 1), lambda i: (i, 0)),
            pl.BlockSpec((D_H, D_H), lambda i: (0, 0)),
            pl.BlockSpec((1, D_H), lambda i: (0, 0)),
        ],
        out_specs=[
            pl.BlockSpec((BLK, D_H), lambda i: (i, 0)),
            pl.BlockSpec((2, D_H), lambda i: (0, 0)),
        ],
        out_shape=[
            jax.ShapeDtypeStruct((N_PAD, D_H), jnp.float32),
            jax.ShapeDtypeStruct((2, D_H), jnp.float32),
        ],
    )(h_in, Sa, Sa, Sb, Sb, rdeg, Ws, b.reshape(1, -1))


def _post_body(split, hpre_ref, stats_ref, g_ref, be_ref, wn_ref, hact_ref, *y_refs):
    mu = stats_ref[0:1, :] * (1.0 / N)
    var = stats_ref[1:2, :] * (1.0 / N) - mu * mu
    rstd = lax.rsqrt(var + 1e-5)
    h = (hpre_ref[...] - mu) * (rstd * g_ref[...]) + be_ref[...]
    h = jnp.maximum(h, 0.0)
    hact_ref[...] = h
    y = jnp.dot(h, wn_ref[...], preferred_element_type=jnp.float32)
    if split:
        y_refs[0][...] = y[:, :64]
        y_refs[1][...] = y[:, 64:]
    else:
        pad = y_refs[0].shape[1] - y.shape[1]
        if pad:
            y = jnp.concatenate([y, jnp.zeros((y.shape[0], pad), jnp.float32)], 1)
        y_refs[0][...] = y


def _post(hpre, stats, g, be, Wn_next, split, dn=None):
    if split:
        outs = [(N_PAD, D_H), (N_PAD, 64), (N_PAD, 64)]
        y_specs = [pl.BlockSpec((BLK, 64), lambda i: (i, 0)) for _ in range(2)]
    else:
        outs = [(N_PAD, D_H), (N_PAD, dn)]
        y_specs = [pl.BlockSpec((BLK, dn), lambda i: (i, 0))]
    return pl.pallas_call(
        functools.partial(_post_body, split),
        grid=(GRID,),
        in_specs=[
            pl.BlockSpec((BLK, D_H), lambda i: (i, 0)),
            pl.BlockSpec((2, D_H), lambda i: (0, 0)),
            pl.BlockSpec((1, D_H), lambda i: (0, 0)),
            pl.BlockSpec((1, D_H), lambda i: (0, 0)),
            pl.BlockSpec(Wn_next.shape, lambda i: (0, 0)),
        ],
        out_specs=[pl.BlockSpec((BLK, D_H), lambda i: (i, 0))] + y_specs,
        out_shape=[jax.ShapeDtypeStruct(o, jnp.float32) for o in outs],
    )(hpre, stats, g.reshape(1, -1), be.reshape(1, -1), Wn_next)


def _final_body(hin_ref, s0, s1, rdeg_ref, ws_ref, b_ref, out_ref):
    ssum = s0[0] + s1[0]
    h = (jnp.dot(hin_ref[...], ws_ref[...], preferred_element_type=jnp.float32)
         + ssum[:, :D_OUT] * rdeg_ref[...] + b_ref[...])
    m = jnp.max(h, axis=1, keepdims=True)
    e = jnp.exp(h - m)
    lse = jnp.log(jnp.sum(e, axis=1, keepdims=True))
    out_ref[...] = h - m - lse


def _final(h_in, S, rdeg, Ws3, b3):
    return pl.pallas_call(
        _final_body,
        grid=(GRID,),
        in_specs=[pl.BlockSpec((BLK, D_H), lambda i: (i, 0))]
        + _plane_specs(48)
        + [
            pl.BlockSpec((BLK, 1), lambda i: (i, 0)),
            pl.BlockSpec((D_H, D_OUT), lambda i: (0, 0)),
            pl.BlockSpec((1, D_OUT), lambda i: (0, 0)),
        ],
        out_specs=pl.BlockSpec((BLK, D_OUT), lambda i: (i, 0)),
        out_shape=jax.ShapeDtypeStruct((N_PAD, D_OUT), jnp.float32),
    )(h_in, S, S, rdeg, Ws3, b3.reshape(1, -1))


def kernel(x, edge_index, Ws1, Wn1, b1, g1, be1, Ws2, Wn2, b2, g2, be2, Ws3, Wn3, b3):
    x_pad = jnp.pad(x, ((0, N_PAD - N), (0, 0)))
    src = edge_index[0]
    dst = edge_index[1]
    src4 = jnp.pad(src, (0, E_PAD - E)).reshape(NTILE, NBLOCK, IB, CHUNK)
    dst4 = jnp.pad(dst, (0, E_PAD - E), constant_values=N).reshape(
        NTILE, NBLOCK, IB, CHUNK)

    y1a, y1b = _mm0(x_pad, Wn1)
    S1a, Sd = _segment_sum(y1a, src4, dst4, with_deg=True)
    S1b = _segment_sum(y1b, src4, dst4)
    hpre1, rdeg, st1 = _pre1(x_pad, S1a, S1b, Sd, Ws1, b1)
    h1, y2a, y2b = _post(hpre1, st1, g1, be1, Wn2, split=True)

    S2a = _segment_sum(y2a, src4, dst4)
    S2b = _segment_sum(y2b, src4, dst4)
    hpre2, st2 = _pre2(h1, S2a, S2b, rdeg, Ws2, b2)
    h2, y3 = _post(hpre2, st2, g2, be2, Wn3, split=False, dn=48)

    S3 = _segment_sum(y3, src4, dst4)
    out = _final(h2, S3, rdeg, Ws3, b3)
    return out[:N, :]


# speedup vs baseline: 4.4004x; 1.0096x over previous
import functools

import jax
import jax.numpy as jnp
from jax import lax
from jax.experimental import pallas as pl
from jax.experimental.pallas import tpu as pltpu
from jax.experimental.pallas import tpu_sc as plsc

N = 10000
E = 320000
D_IN = 128
D_H = 128
D_OUT = 40
DDEG = 16

N_PAD = 10048
RPT = N_PAD // 16
CHUNK = 64
NTILE = 32
NBLOCK = 10
IB = 16
NBUF = 16
NCH = NBLOCK * IB
E_PAD = NTILE * NCH * CHUNK

BLK = 1256
GRID = N_PAD // BLK


def _sc_scatter_fn(D, with_deg):
    mesh = plsc.VectorSubcoreMesh(core_axis_name="c", subcore_axis_name="s")

    out_type = [jax.ShapeDtypeStruct((2, N_PAD, D), jnp.float32)]
    scratch = [
        pltpu.VMEM((2, IB, CHUNK), jnp.int32),
        pltpu.VMEM((2, IB, CHUNK), jnp.int32),
        [pltpu.VMEM((CHUNK, D), jnp.float32) for _ in range(NBUF)],
        pltpu.VMEM_SHARED((N_PAD, D), jnp.float32),
        [pltpu.SemaphoreType.DMA for _ in range(NBUF)],
    ]
    if with_deg:
        out_type.append(jax.ShapeDtypeStruct((2, N_PAD, DDEG), jnp.float32))
        scratch += [
            pltpu.VMEM((CHUNK, DDEG), jnp.float32),
            pltpu.VMEM_SHARED((N_PAD, DDEG), jnp.float32),
        ]

    @functools.partial(
        pl.kernel,
        out_type=out_type,
        mesh=mesh,
        compiler_params=pltpu.CompilerParams(use_tc_tiling_on_sc=False),
        scratch_types=scratch,
    )
    def sc_scatter(y_hbm, src_hbm, dst_hbm, zero_hbm, *rest):
        if with_deg:
            (ones_hbm, zdeg_hbm, out_hbm, deg_hbm,
             src_v, dst_v, rows, acc_sh, sems, ones_v, dacc_sh) = rest
        else:
            out_hbm, src_v, dst_v, rows, acc_sh, sems = rest
        c = lax.axis_index("c")
        s = lax.axis_index("s")
        tile = c * 16 + s
        pltpu.sync_copy(zero_hbm, acc_sh.at[pl.ds(s * RPT, RPT)])
        if with_deg:
            pltpu.sync_copy(zdeg_hbm, dacc_sh.at[pl.ds(s * RPT, RPT)])
            pltpu.sync_copy(ones_hbm, ones_v)
        plsc.subcore_barrier()

        def scat(b, didx):
            pltpu.sync_copy(rows[b], acc_sh.at[didx], add=True)
            if with_deg:
                pltpu.sync_copy(ones_v, dacc_sh.at[didx], add=True)

        pltpu.sync_copy(src_hbm.at[tile, 0], src_v.at[0])
        pltpu.sync_copy(dst_hbm.at[tile, 0], dst_v.at[0])
        for b in range(NBUF):
            pltpu.async_copy(y_hbm.at[src_v.at[0, b]], rows[b], sems[b])

        @pl.loop(0, NBLOCK)
        def _(bi):
            cur = lax.rem(bi, 2)
            nxt = 1 - cur

            @pl.when(bi + 1 < NBLOCK)
            def _():
                pltpu.sync_copy(src_hbm.at[tile, bi + 1], src_v.at[nxt])
                pltpu.sync_copy(dst_hbm.at[tile, bi + 1], dst_v.at[nxt])

            if IB > NBUF:
                @pl.loop(0, IB - NBUF, step=NBUF)
                def _(j):
                    for b in range(NBUF):
                        pltpu.make_async_copy(
                            y_hbm.at[src_v.at[0, 0]], rows[b], sems[b]).wait()
                        scat(b, dst_v.at[cur, j + b])
                        pltpu.async_copy(
                            y_hbm.at[src_v.at[cur, j + b + NBUF]], rows[b], sems[b])

            for b in range(NBUF):
                pltpu.make_async_copy(
                    y_hbm.at[src_v.at[0, 0]], rows[b], sems[b]).wait()
                scat(b, dst_v.at[cur, IB - NBUF + b])

                @pl.when(bi + 1 < NBLOCK)
                def _():
                    pltpu.async_copy(y_hbm.at[src_v.at[nxt, b]], rows[b], sems[b])

        plsc.subcore_barrier()
        pltpu.sync_copy(acc_sh.at[pl.ds(s * RPT, RPT)],
                        out_hbm.at[c, pl.ds(s * RPT, RPT)])
        if with_deg:
            pltpu.sync_copy(dacc_sh.at[pl.ds(s * RPT, RPT)],
                            deg_hbm.at[c, pl.ds(s * RPT, RPT)])

    return sc_scatter


def _segment_sum(y, src4, dst4, with_deg=False):
    D = y.shape[1]
    zero = jnp.zeros((RPT, D), jnp.float32)
    if with_deg:
        ones = jnp.ones((CHUNK, DDEG), jnp.float32)
        zdeg = jnp.zeros((RPT, DDEG), jnp.float32)
        out = _sc_scatter_fn(D, True)(y, src4, dst4, zero, ones, zdeg)
        return out[0], out[1]
    out = _sc_scatter_fn(D, False)(y, src4, dst4, zero)
    return out[0] if isinstance(out, (list, tuple)) else out


def _mm0_body(x_ref, wn_ref, ya_ref, yb_ref):
    y = jnp.dot(x_ref[...], wn_ref[...], preferred_element_type=jnp.float32)
    ya_ref[...] = y[:, :64]
    yb_ref[...] = y[:, 64:]


def _mm0(x_pad, Wn1):
    return pl.pallas_call(
        _mm0_body,
        grid=(GRID,),
        in_specs=[
            pl.BlockSpec((BLK, D_IN), lambda i: (i, 0)),
            pl.BlockSpec((D_IN, D_H), lambda i: (0, 0)),
        ],
        out_specs=[
            pl.BlockSpec((BLK, 64), lambda i: (i, 0)),
            pl.BlockSpec((BLK, 64), lambda i: (i, 0)),
        ],
        out_shape=[
            jax.ShapeDtypeStruct((N_PAD, 64), jnp.float32),
            jax.ShapeDtypeStruct((N_PAD, 64), jnp.float32),
        ],
    )(x_pad, Wn1)


def _stats_update(i, h, stats_ref):
    rows = i * BLK + lax.broadcasted_iota(jnp.int32, (BLK, 1), 0)
    hm = jnp.where(rows < N, h, 0.0)
    st = jnp.stack([jnp.sum(hm, axis=0), jnp.sum(hm * hm, axis=0)])

    @pl.when(i == 0)
    def _():
        stats_ref[...] = st

    @pl.when(i > 0)
    def _():
        stats_ref[...] += st


def _pre1_body(x_ref, sa0, sa1, sb0, sb1, sd0, sd1, ws_ref, b_ref,
               h_ref, rdeg_ref, stats_ref):
    i = pl.program_id(0)
    ssum = jnp.concatenate([sa0[0] + sa1[0], sb0[0] + sb1[0]], axis=1)
    deg = (sd0[0] + sd1[0])[:, 0:1]
    rdeg = 1.0 / jnp.maximum(deg, 1.0)
    h = (jnp.dot(x_ref[...], ws_ref[...], preferred_element_type=jnp.float32)
         + ssum * rdeg + b_ref[...])
    h_ref[...] = h
    rdeg_ref[...] = rdeg
    _stats_update(i, h, stats_ref)


def _plane_specs(cols):
    return [pl.BlockSpec((1, BLK, cols), lambda i: (0, i, 0)),
            pl.BlockSpec((1, BLK, cols), lambda i: (1, i, 0))]


def _pre1(x_pad, Sa, Sb, Sd, Ws1, b1):
    return pl.pallas_call(
        _pre1_body,
        grid=(GRID,),
        in_specs=[pl.BlockSpec((BLK, D_IN), lambda i: (i, 0))]
        + _plane_specs(64) + _plane_specs(64) + _plane_specs(DDEG)
        + [
            pl.BlockSpec((D_IN, D_H), lambda i: (0, 0)),
            pl.BlockSpec((1, D_H), lambda i: (0, 0)),
        ],
        out_specs=[
            pl.BlockSpec((BLK, D_H), lambda i: (i, 0)),
            pl.BlockSpec((BLK, 1), lambda i: (i, 0)),
            pl.BlockSpec((2, D_H), lambda i: (0, 0)),
        ],
        out_shape=[
            jax.ShapeDtypeStruct((N_PAD, D_H), jnp.float32),
            jax.ShapeDtypeStruct((N_PAD, 1), jnp.float32),
            jax.ShapeDtypeStruct((2, D_H), jnp.float32),
        ],
    )(x_pad, Sa, Sa, Sb, Sb, Sd, Sd, Ws1, b1.reshape(1, -1))


def _pre2_body(hin_ref, sa0, sa1, sb0, sb1, rdeg_ref, ws_ref, b_ref,
               h_ref, stats_ref):
    i = pl.program_id(0)
    ssum = jnp.concatenate([sa0[0] + sa1[0], sb0[0] + sb1[0]], axis=1)
    h = (jnp.dot(hin_ref[...], ws_ref[...], preferred_element_type=jnp.float32)
         + ssum * rdeg_ref[...] + b_ref[...])
    h_ref[...] = h
    _stats_update(i, h, stats_ref)


def _pre2(h_in, Sa, Sb, rdeg, Ws, b):
    return pl.pallas_call(
        _pre2_body,
        grid=(GRID,),
        in_specs=[pl.BlockSpec((BLK, D_H), lambda i: (i, 0))]
        + _plane_specs(64) + _plane_specs(64)
        + [
            pl.BlockSpec((BLK, 1), lambda i: (i, 0)),
            pl.BlockSpec((D_H, D_H), lambda i: (0, 0)),
            pl.BlockSpec((1, D_H), lambda i: (0, 0)),
        ],
        out_specs=[
            pl.BlockSpec((BLK, D_H), lambda i: (i, 0)),
            pl.BlockSpec((2, D_H), lambda i: (0, 0)),
        ],
        out_shape=[
            jax.ShapeDtypeStruct((N_PAD, D_H), jnp.float32),
            jax.ShapeDtypeStruct((2, D_H), jnp.float32),
        ],
    )(h_in, Sa, Sa, Sb, Sb, rdeg, Ws, b.reshape(1, -1))


def _post_body(split, hpre_ref, stats_ref, g_ref, be_ref, wn_ref, hact_ref, *y_refs):
    mu = stats_ref[0:1, :] * (1.0 / N)
    var = stats_ref[1:2, :] * (1.0 / N) - mu * mu
    rstd = lax.rsqrt(var + 1e-5)
    h = (hpre_ref[...] - mu) * (rstd * g_ref[...]) + be_ref[...]
    h = jnp.maximum(h, 0.0)
    hact_ref[...] = h
    y = jnp.dot(h, wn_ref[...], preferred_element_type=jnp.float32)
    if split:
        y_refs[0][...] = y[:, :64]
        y_refs[1][...] = y[:, 64:]
    else:
        pad = y_refs[0].shape[1] - y.shape[1]
        if pad:
            y = jnp.concatenate([y, jnp.zeros((y.shape[0], pad), jnp.float32)], 1)
        y_refs[0][...] = y


def _post(hpre, stats, g, be, Wn_next, split, dn=None):
    if split:
        outs = [(N_PAD, D_H), (N_PAD, 64), (N_PAD, 64)]
        y_specs = [pl.BlockSpec((BLK, 64), lambda i: (i, 0)) for _ in range(2)]
    else:
        outs = [(N_PAD, D_H), (N_PAD, dn)]
        y_specs = [pl.BlockSpec((BLK, dn), lambda i: (i, 0))]
    return pl.pallas_call(
        functools.partial(_post_body, split),
        grid=(GRID,),
        in_specs=[
            pl.BlockSpec((BLK, D_H), lambda i: (i, 0)),
            pl.BlockSpec((2, D_H), lambda i: (0, 0)),
            pl.BlockSpec((1, D_H), lambda i: (0, 0)),
            pl.BlockSpec((1, D_H), lambda i: (0, 0)),
            pl.BlockSpec(Wn_next.shape, lambda i: (0, 0)),
        ],
        out_specs=[pl.BlockSpec((BLK, D_H), lambda i: (i, 0))] + y_specs,
        out_shape=[jax.ShapeDtypeStruct(o, jnp.float32) for o in outs],
    )(hpre, stats, g.reshape(1, -1), be.reshape(1, -1), Wn_next)


def _final_body(hin_ref, s0, s1, rdeg_ref, ws_ref, b_ref, out_ref):
    ssum = s0[0] + s1[0]
    h = (jnp.dot(hin_ref[...], ws_ref[...], preferred_element_type=jnp.float32)
         + ssum[:, :D_OUT] * rdeg_ref[...] + b_ref[...])
    m = jnp.max(h, axis=1, keepdims=True)
    e = jnp.exp(h - m)
    lse = jnp.log(jnp.sum(e, axis=1, keepdims=True))
    out_ref[...] = h - m - lse


def _final(h_in, S, rdeg, Ws3, b3):
    return pl.pallas_call(
        _final_body,
        grid=(GRID,),
        in_specs=[pl.BlockSpec((BLK, D_H), lambda i: (i, 0))]
        + _plane_specs(48)
        + [
            pl.BlockSpec((BLK, 1), lambda i: (i, 0)),
            pl.BlockSpec((D_H, D_OUT), lambda i: (0, 0)),
            pl.BlockSpec((1, D_OUT), lambda i: (0, 0)),
        ],
        out_specs=pl.BlockSpec((BLK, D_OUT), lambda i: (i, 0)),
        out_shape=jax.ShapeDtypeStruct((N_PAD, D_OUT), jnp.float32),
    )(h_in, S, S, rdeg, Ws3, b3.reshape(1, -1))


def kernel(x, edge_index, Ws1, Wn1, b1, g1, be1, Ws2, Wn2, b2, g2, be2, Ws3, Wn3, b3):
    x_pad = jnp.pad(x, ((0, N_PAD - N), (0, 0)))
    src = edge_index[0]
    dst = edge_index[1]
    src4 = jnp.pad(src, (0, E_PAD - E)).reshape(NTILE, NBLOCK, IB, CHUNK)
    dst4 = jnp.pad(dst, (0, E_PAD - E), constant_values=N).reshape(
        NTILE, NBLOCK, IB, CHUNK)

    y1a, y1b = _mm0(x_pad, Wn1)
    S1a, Sd = _segment_sum(y1a, src4, dst4, with_deg=True)
    S1b = _segment_sum(y1b, src4, dst4)
    hpre1, rdeg, st1 = _pre1(x_pad, S1a, S1b, Sd, Ws1, b1)
    h1, y2a, y2b = _post(hpre1, st1, g1, be1, Wn2, split=True)

    S2a = _segment_sum(y2a, src4, dst4)
    S2b = _segment_sum(y2b, src4, dst4)
    hpre2, st2 = _pre2(h1, S2a, S2b, rdeg, Ws2, b2)
    h2, y3 = _post(hpre2, st2, g2, be2, Wn3, split=False, dn=48)

    S3 = _segment_sum(y3, src4, dst4)
    out = _final(h2, S3, rdeg, Ws3, b3)
    return out[:N, :]


# speedup vs baseline: 8.5404x; 1.9408x over previous
import functools

import jax
import jax.numpy as jnp
from jax import lax
from jax.experimental import pallas as pl
from jax.experimental.pallas import tpu as pltpu
from jax.experimental.pallas import tpu_sc as plsc

N = 10000
E = 320000
D_IN = 128
D_H = 128
D_OUT = 40
DDEG = 16

N_PAD = 10048
RPT = N_PAD // 16
CHUNK = 64
NTILE = 32
NBLOCK = 20
IB = 8
NBUF = 8
NCH = NBLOCK * IB
E_PAD = NTILE * NCH * CHUNK

BLK = 1256
GRID = N_PAD // BLK


def _sc_scatter_fn(D, with_deg):
    mesh = plsc.VectorSubcoreMesh(core_axis_name="c", subcore_axis_name="s")

    out_type = [jax.ShapeDtypeStruct((2, N_PAD, D), jnp.float32)]
    scratch = [
        pltpu.VMEM((2, IB, CHUNK), jnp.int32),
        pltpu.VMEM((2, IB, CHUNK), jnp.int32),
        [pltpu.VMEM((CHUNK, D), jnp.float32) for _ in range(NBUF)],
        pltpu.VMEM_SHARED((N_PAD, D), jnp.float32),
        pltpu.VMEM_SHARED((N_PAD, D), jnp.float32),
        [pltpu.SemaphoreType.DMA for _ in range(NBUF)],
    ]
    if with_deg:
        out_type.append(jax.ShapeDtypeStruct((2, N_PAD, DDEG), jnp.float32))
        scratch += [
            pltpu.VMEM((CHUNK, DDEG), jnp.float32),
            pltpu.VMEM_SHARED((N_PAD, DDEG), jnp.float32),
        ]

    @functools.partial(
        pl.kernel,
        out_type=out_type,
        mesh=mesh,
        compiler_params=pltpu.CompilerParams(use_tc_tiling_on_sc=False),
        scratch_types=scratch,
    )
    def sc_scatter(y_hbm, src_hbm, dst_hbm, zero_hbm, *rest):
        if with_deg:
            (ones_hbm, zdeg_hbm, out_hbm, deg_hbm,
             src_v, dst_v, rows, acc_sh, y_sh, sems, ones_v, dacc_sh) = rest
        else:
            out_hbm, src_v, dst_v, rows, acc_sh, y_sh, sems = rest
        c = lax.axis_index("c")
        s = lax.axis_index("s")
        tile = c * 16 + s
        pltpu.sync_copy(zero_hbm, acc_sh.at[pl.ds(s * RPT, RPT)])
        pltpu.sync_copy(y_hbm.at[pl.ds(s * RPT, RPT)], y_sh.at[pl.ds(s * RPT, RPT)])
        if with_deg:
            pltpu.sync_copy(zdeg_hbm, dacc_sh.at[pl.ds(s * RPT, RPT)])
            pltpu.sync_copy(ones_hbm, ones_v)
        plsc.subcore_barrier()

        def scat(b, didx):
            pltpu.sync_copy(rows[b], acc_sh.at[didx], add=True)
            if with_deg:
                pltpu.sync_copy(ones_v, dacc_sh.at[didx], add=True)

        pltpu.sync_copy(src_hbm.at[tile, 0], src_v.at[0])
        pltpu.sync_copy(dst_hbm.at[tile, 0], dst_v.at[0])
        for b in range(NBUF):
            pltpu.async_copy(y_sh.at[src_v.at[0, b]], rows[b], sems[b])

        @pl.loop(0, NBLOCK)
        def _(bi):
            cur = lax.rem(bi, 2)
            nxt = 1 - cur

            @pl.when(bi + 1 < NBLOCK)
            def _():
                pltpu.sync_copy(src_hbm.at[tile, bi + 1], src_v.at[nxt])
                pltpu.sync_copy(dst_hbm.at[tile, bi + 1], dst_v.at[nxt])

            if IB > NBUF:
                @pl.loop(0, IB - NBUF, step=NBUF)
                def _(j):
                    for b in range(NBUF):
                        pltpu.make_async_copy(
                            y_sh.at[src_v.at[0, 0]], rows[b], sems[b]).wait()
                        scat(b, dst_v.at[cur, j + b])
                        pltpu.async_copy(
                            y_sh.at[src_v.at[cur, j + b + NBUF]], rows[b], sems[b])

            for b in range(NBUF):
                pltpu.make_async_copy(
                    y_sh.at[src_v.at[0, 0]], rows[b], sems[b]).wait()
                scat(b, dst_v.at[cur, IB - NBUF + b])

                @pl.when(bi + 1 < NBLOCK)
                def _():
                    pltpu.async_copy(y_sh.at[src_v.at[nxt, b]], rows[b], sems[b])

        plsc.subcore_barrier()
        pltpu.sync_copy(acc_sh.at[pl.ds(s * RPT, RPT)],
                        out_hbm.at[c, pl.ds(s * RPT, RPT)])
        if with_deg:
            pltpu.sync_copy(dacc_sh.at[pl.ds(s * RPT, RPT)],
                            deg_hbm.at[c, pl.ds(s * RPT, RPT)])

    return sc_scatter


def _segment_sum(y, src4, dst4, with_deg=False):
    D = y.shape[1]
    zero = jnp.zeros((RPT, D), jnp.float32)
    if with_deg:
        ones = jnp.ones((CHUNK, DDEG), jnp.float32)
        zdeg = jnp.zeros((RPT, DDEG), jnp.float32)
        out = _sc_scatter_fn(D, True)(y, src4, dst4, zero, ones, zdeg)
        return out[0], out[1]
    out = _sc_scatter_fn(D, False)(y, src4, dst4, zero)
    return out[0] if isinstance(out, (list, tuple)) else out


def _mm0_body(x_ref, wn_ref, ya_ref, yb_ref):
    y = jnp.dot(x_ref[...], wn_ref[...], preferred_element_type=jnp.float32)
    ya_ref[...] = y[:, :64]
    yb_ref[...] = y[:, 64:]


def _mm0(x_pad, Wn1):
    return pl.pallas_call(
        _mm0_body,
        grid=(GRID,),
        in_specs=[
            pl.BlockSpec((BLK, D_IN), lambda i: (i, 0)),
            pl.BlockSpec((D_IN, D_H), lambda i: (0, 0)),
        ],
        out_specs=[
            pl.BlockSpec((BLK, 64), lambda i: (i, 0)),
            pl.BlockSpec((BLK, 64), lambda i: (i, 0)),
        ],
        out_shape=[
            jax.ShapeDtypeStruct((N_PAD, 64), jnp.float32),
            jax.ShapeDtypeStruct((N_PAD, 64), jnp.float32),
        ],
    )(x_pad, Wn1)


def _stats_update(i, h, stats_ref):
    rows = i * BLK + lax.broadcasted_iota(jnp.int32, (BLK, 1), 0)
    hm = jnp.where(rows < N, h, 0.0)
    st = jnp.stack([jnp.sum(hm, axis=0), jnp.sum(hm * hm, axis=0)])

    @pl.when(i == 0)
    def _():
        stats_ref[...] = st

    @pl.when(i > 0)
    def _():
        stats_ref[...] += st


def _pre1_body(x_ref, sa0, sa1, sb0, sb1, sd0, sd1, ws_ref, b_ref,
               h_ref, rdeg_ref, stats_ref):
    i = pl.program_id(0)
    ssum = jnp.concatenate([sa0[0] + sa1[0], sb0[0] + sb1[0]], axis=1)
    deg = (sd0[0] + sd1[0])[:, 0:1]
    rdeg = 1.0 / jnp.maximum(deg, 1.0)
    h = (jnp.dot(x_ref[...], ws_ref[...], preferred_element_type=jnp.float32)
         + ssum * rdeg + b_ref[...])
    h_ref[...] = h
    rdeg_ref[...] = rdeg
    _stats_update(i, h, stats_ref)


def _plane_specs(cols):
    return [pl.BlockSpec((1, BLK, cols), lambda i: (0, i, 0)),
            pl.BlockSpec((1, BLK, cols), lambda i: (1, i, 0))]


def _pre1(x_pad, Sa, Sb, Sd, Ws1, b1):
    return pl.pallas_call(
        _pre1_body,
        grid=(GRID,),
        in_specs=[pl.BlockSpec((BLK, D_IN), lambda i: (i, 0))]
        + _plane_specs(64) + _plane_specs(64) + _plane_specs(DDEG)
        + [
            pl.BlockSpec((D_IN, D_H), lambda i: (0, 0)),
            pl.BlockSpec((1, D_H), lambda i: (0, 0)),
        ],
        out_specs=[
            pl.BlockSpec((BLK, D_H), lambda i: (i, 0)),
            pl.BlockSpec((BLK, 1), lambda i: (i, 0)),
            pl.BlockSpec((2, D_H), lambda i: (0, 0)),
        ],
        out_shape=[
            jax.ShapeDtypeStruct((N_PAD, D_H), jnp.float32),
            jax.ShapeDtypeStruct((N_PAD, 1), jnp.float32),
            jax.ShapeDtypeStruct((2, D_H), jnp.float32),
        ],
    )(x_pad, Sa, Sa, Sb, Sb, Sd, Sd, Ws1, b1.reshape(1, -1))


def _pre2_body(hin_ref, sa0, sa1, sb0, sb1, rdeg_ref, ws_ref, b_ref,
               h_ref, stats_ref):
    i = pl.program_id(0)
    ssum = jnp.concatenate([sa0[0] + sa1[0], sb0[0] + sb1[0]], axis=1)
    h = (jnp.dot(hin_ref[...], ws_ref[...], preferred_element_type=jnp.float32)
         + ssum * rdeg_ref[...] + b_ref[...])
    h_ref[...] = h
    _stats_update(i, h, stats_ref)


def _pre2(h_in, Sa, Sb, rdeg, Ws, b):
    return pl.pallas_call(
        _pre2_body,
        grid=(GRID,),
        in_specs=[pl.BlockSpec((BLK, D_H), lambda i: (i, 0))]
        + _plane_specs(64) + _plane_specs(64)
        + [
            pl.BlockSpec((BLK, 1), lambda i: (i, 0)),
            pl.BlockSpec((D_H, D_H), lambda i: (0, 0)),
            pl.BlockSpec((1, D_H), lambda i: (0, 0)),
        ],
        out_specs=[
            pl.BlockSpec((BLK, D_H), lambda i: (i, 0)),
            pl.BlockSpec((2, D_H), lambda i: (0, 0)),
        ],
        out_shape=[
            jax.ShapeDtypeStruct((N_PAD, D_H), jnp.float32),
            jax.ShapeDtypeStruct((2, D_H), jnp.float32),
        ],
    )(h_in, Sa, Sa, Sb, Sb, rdeg, Ws, b.reshape(1, -1))


def _post_body(split, hpre_ref, stats_ref, g_ref, be_ref, wn_ref, hact_ref, *y_refs):
    mu = stats_ref[0:1, :] * (1.0 / N)
    var = stats_ref[1:2, :] * (1.0 / N) - mu * mu
    rstd = lax.rsqrt(var + 1e-5)
    h = (hpre_ref[...] - mu) * (rstd * g_ref[...]) + be_ref[...]
    h = jnp.maximum(h, 0.0)
    hact_ref[...] = h
    y = jnp.dot(h, wn_ref[...], preferred_element_type=jnp.float32)
    if split:
        y_refs[0][...] = y[:, :64]
        y_refs[1][...] = y[:, 64:]
    else:
        pad = y_refs[0].shape[1] - y.shape[1]
        if pad:
            y = jnp.concatenate([y, jnp.zeros((y.shape[0], pad), jnp.float32)], 1)
        y_refs[0][...] = y


def _post(hpre, stats, g, be, Wn_next, split, dn=None):
    if split:
        outs = [(N_PAD, D_H), (N_PAD, 64), (N_PAD, 64)]
        y_specs = [pl.BlockSpec((BLK, 64), lambda i: (i, 0)) for _ in range(2)]
    else:
        outs = [(N_PAD, D_H), (N_PAD, dn)]
        y_specs = [pl.BlockSpec((BLK, dn), lambda i: (i, 0))]
    return pl.pallas_call(
        functools.partial(_post_body, split),
        grid=(GRID,),
        in_specs=[
            pl.BlockSpec((BLK, D_H), lambda i: (i, 0)),
            pl.BlockSpec((2, D_H), lambda i: (0, 0)),
            pl.BlockSpec((1, D_H), lambda i: (0, 0)),
            pl.BlockSpec((1, D_H), lambda i: (0, 0)),
            pl.BlockSpec(Wn_next.shape, lambda i: (0, 0)),
        ],
        out_specs=[pl.BlockSpec((BLK, D_H), lambda i: (i, 0))] + y_specs,
        out_shape=[jax.ShapeDtypeStruct(o, jnp.float32) for o in outs],
    )(hpre, stats, g.reshape(1, -1), be.reshape(1, -1), Wn_next)


def _final_body(hin_ref, s0, s1, rdeg_ref, ws_ref, b_ref, out_ref):
    ssum = s0[0] + s1[0]
    h = (jnp.dot(hin_ref[...], ws_ref[...], preferred_element_type=jnp.float32)
         + ssum[:, :D_OUT] * rdeg_ref[...] + b_ref[...])
    m = jnp.max(h, axis=1, keepdims=True)
    e = jnp.exp(h - m)
    lse = jnp.log(jnp.sum(e, axis=1, keepdims=True))
    out_ref[...] = h - m - lse


def _final(h_in, S, rdeg, Ws3, b3):
    return pl.pallas_call(
        _final_body,
        grid=(GRID,),
        in_specs=[pl.BlockSpec((BLK, D_H), lambda i: (i, 0))]
        + _plane_specs(48)
        + [
            pl.BlockSpec((BLK, 1), lambda i: (i, 0)),
            pl.BlockSpec((D_H, D_OUT), lambda i: (0, 0)),
            pl.BlockSpec((1, D_OUT), lambda i: (0, 0)),
        ],
        out_specs=pl.BlockSpec((BLK, D_OUT), lambda i: (i, 0)),
        out_shape=jax.ShapeDtypeStruct((N_PAD, D_OUT), jnp.float32),
    )(h_in, S, S, rdeg, Ws3, b3.reshape(1, -1))


def kernel(x, edge_index, Ws1, Wn1, b1, g1, be1, Ws2, Wn2, b2, g2, be2, Ws3, Wn3, b3):
    x_pad = jnp.pad(x, ((0, N_PAD - N), (0, 0)))
    src = edge_index[0]
    dst = edge_index[1]
    src4 = jnp.pad(src, (0, E_PAD - E)).reshape(NTILE, NBLOCK, IB, CHUNK)
    dst4 = jnp.pad(dst, (0, E_PAD - E), constant_values=N).reshape(
        NTILE, NBLOCK, IB, CHUNK)

    y1a, y1b = _mm0(x_pad, Wn1)
    S1a, Sd = _segment_sum(y1a, src4, dst4, with_deg=True)
    S1b = _segment_sum(y1b, src4, dst4)
    hpre1, rdeg, st1 = _pre1(x_pad, S1a, S1b, Sd, Ws1, b1)
    h1, y2a, y2b = _post(hpre1, st1, g1, be1, Wn2, split=True)

    S2a = _segment_sum(y2a, src4, dst4)
    S2b = _segment_sum(y2b, src4, dst4)
    hpre2, st2 = _pre2(h1, S2a, S2b, rdeg, Ws2, b2)
    h2, y3 = _post(hpre2, st2, g2, be2, Wn3, split=False, dn=48)

    S3 = _segment_sum(y3, src4, dst4)
    out = _final(h2, S3, rdeg, Ws3, b3)
    return out[:N, :]


# speedup vs baseline: 8.6733x; 1.0156x over previous
import functools

import jax
import jax.numpy as jnp
from jax import lax
from jax.experimental import pallas as pl
from jax.experimental.pallas import tpu as pltpu
from jax.experimental.pallas import tpu_sc as plsc

N = 10000
E = 320000
D_IN = 128
D_H = 128
D_OUT = 40
DDEG = 16

N_PAD = 10048
RPT = N_PAD // 16
CHUNK = 64
NTILE = 32
NBLOCK = 20
IB = 8
NBUF = 8
NCH = NBLOCK * IB
E_PAD = NTILE * NCH * CHUNK

BLK = 1256
GRID = N_PAD // BLK


def _sc_scatter_fn(D, with_deg):
    mesh = plsc.VectorSubcoreMesh(core_axis_name="c", subcore_axis_name="s")

    out_type = [jax.ShapeDtypeStruct((2, N_PAD, D), jnp.float32)]
    scratch = [
        pltpu.VMEM((2, IB, CHUNK), jnp.int32),
        pltpu.VMEM((2, IB, CHUNK), jnp.int32),
        [pltpu.VMEM((CHUNK, D), jnp.float32) for _ in range(NBUF)],
        pltpu.VMEM_SHARED((N_PAD, D), jnp.float32),
        pltpu.VMEM_SHARED((N_PAD, D), jnp.float32),
        [pltpu.SemaphoreType.DMA for _ in range(NBUF)],
    ]
    if with_deg:
        out_type.append(jax.ShapeDtypeStruct((2, N_PAD, DDEG), jnp.float32))
        scratch += [
            pltpu.VMEM((CHUNK, DDEG), jnp.float32),
            pltpu.VMEM_SHARED((N_PAD, DDEG), jnp.float32),
        ]

    @functools.partial(
        pl.kernel,
        out_type=out_type,
        mesh=mesh,
        compiler_params=pltpu.CompilerParams(use_tc_tiling_on_sc=False),
        scratch_types=scratch,
    )
    def sc_scatter(y_hbm, src_hbm, dst_hbm, zero_hbm, *rest):
        if with_deg:
            (ones_hbm, zdeg_hbm, out_hbm, deg_hbm,
             src_v, dst_v, rows, acc_sh, y_sh, sems, ones_v, dacc_sh) = rest
        else:
            out_hbm, src_v, dst_v, rows, acc_sh, y_sh, sems = rest
        c = lax.axis_index("c")
        s = lax.axis_index("s")
        tile = c * 16 + s
        pltpu.async_copy(zero_hbm, acc_sh.at[pl.ds(s * RPT, RPT)], sems[0])
        pltpu.async_copy(y_hbm.at[pl.ds(s * RPT, RPT)],
                         y_sh.at[pl.ds(s * RPT, RPT)], sems[1])
        if with_deg:
            pltpu.async_copy(zdeg_hbm, dacc_sh.at[pl.ds(s * RPT, RPT)], sems[2])
            pltpu.async_copy(ones_hbm, ones_v, sems[3])
        pltpu.sync_copy(src_hbm.at[tile, 0], src_v.at[0])
        pltpu.sync_copy(dst_hbm.at[tile, 0], dst_v.at[0])
        pltpu.make_async_copy(zero_hbm, acc_sh.at[pl.ds(s * RPT, RPT)],
                              sems[0]).wait()
        pltpu.make_async_copy(y_hbm.at[pl.ds(s * RPT, RPT)],
                              y_sh.at[pl.ds(s * RPT, RPT)], sems[1]).wait()
        if with_deg:
            pltpu.make_async_copy(zdeg_hbm, dacc_sh.at[pl.ds(s * RPT, RPT)],
                                  sems[2]).wait()
            pltpu.make_async_copy(ones_hbm, ones_v, sems[3]).wait()
        plsc.subcore_barrier()

        def scat(b, didx):
            pltpu.sync_copy(rows[b], acc_sh.at[didx], add=True)
            if with_deg:
                pltpu.sync_copy(ones_v, dacc_sh.at[didx], add=True)

        for b in range(NBUF):
            pltpu.async_copy(y_sh.at[src_v.at[0, b]], rows[b], sems[b])

        @pl.loop(0, NBLOCK)
        def _(bi):
            cur = lax.rem(bi, 2)
            nxt = 1 - cur

            @pl.when(bi + 1 < NBLOCK)
            def _():
                pltpu.sync_copy(src_hbm.at[tile, bi + 1], src_v.at[nxt])
                pltpu.sync_copy(dst_hbm.at[tile, bi + 1], dst_v.at[nxt])

            if IB > NBUF:
                @pl.loop(0, IB - NBUF, step=NBUF)
                def _(j):
                    for b in range(NBUF):
                        pltpu.make_async_copy(
                            y_sh.at[src_v.at[0, 0]], rows[b], sems[b]).wait()
                        scat(b, dst_v.at[cur, j + b])
                        pltpu.async_copy(
                            y_sh.at[src_v.at[cur, j + b + NBUF]], rows[b], sems[b])

            for b in range(NBUF):
                pltpu.make_async_copy(
                    y_sh.at[src_v.at[0, 0]], rows[b], sems[b]).wait()
                scat(b, dst_v.at[cur, IB - NBUF + b])

                @pl.when(bi + 1 < NBLOCK)
                def _():
                    pltpu.async_copy(y_sh.at[src_v.at[nxt, b]], rows[b], sems[b])

        plsc.subcore_barrier()
        pltpu.async_copy(acc_sh.at[pl.ds(s * RPT, RPT)],
                         out_hbm.at[c, pl.ds(s * RPT, RPT)], sems[0])
        if with_deg:
            pltpu.async_copy(dacc_sh.at[pl.ds(s * RPT, RPT)],
                             deg_hbm.at[c, pl.ds(s * RPT, RPT)], sems[1])
            pltpu.make_async_copy(dacc_sh.at[pl.ds(s * RPT, RPT)],
                                  deg_hbm.at[c, pl.ds(s * RPT, RPT)],
                                  sems[1]).wait()
        pltpu.make_async_copy(acc_sh.at[pl.ds(s * RPT, RPT)],
                              out_hbm.at[c, pl.ds(s * RPT, RPT)], sems[0]).wait()

    return sc_scatter


def _segment_sum(y, src4, dst4, with_deg=False):
    D = y.shape[1]
    zero = jnp.zeros((RPT, D), jnp.float32)
    if with_deg:
        ones = jnp.ones((CHUNK, DDEG), jnp.float32)
        zdeg = jnp.zeros((RPT, DDEG), jnp.float32)
        out = _sc_scatter_fn(D, True)(y, src4, dst4, zero, ones, zdeg)
        return out[0], out[1]
    out = _sc_scatter_fn(D, False)(y, src4, dst4, zero)
    return out[0] if isinstance(out, (list, tuple)) else out


def _mm0_body(x_ref, wn_ref, ya_ref, yb_ref):
    y = jnp.dot(x_ref[...], wn_ref[...], preferred_element_type=jnp.float32)
    ya_ref[...] = y[:, :64]
    yb_ref[...] = y[:, 64:]


def _mm0(x_pad, Wn1):
    return pl.pallas_call(
        _mm0_body,
        grid=(GRID,),
        in_specs=[
            pl.BlockSpec((BLK, D_IN), lambda i: (i, 0)),
            pl.BlockSpec((D_IN, D_H), lambda i: (0, 0)),
        ],
        out_specs=[
            pl.BlockSpec((BLK, 64), lambda i: (i, 0)),
            pl.BlockSpec((BLK, 64), lambda i: (i, 0)),
        ],
        out_shape=[
            jax.ShapeDtypeStruct((N_PAD, 64), jnp.float32),
            jax.ShapeDtypeStruct((N_PAD, 64), jnp.float32),
        ],
    )(x_pad, Wn1)


def _stats_update(i, h, stats_ref):
    rows = i * BLK + lax.broadcasted_iota(jnp.int32, (BLK, 1), 0)
    hm = jnp.where(rows < N, h, 0.0)
    st = jnp.stack([jnp.sum(hm, axis=0), jnp.sum(hm * hm, axis=0)])

    @pl.when(i == 0)
    def _():
        stats_ref[...] = st

    @pl.when(i > 0)
    def _():
        stats_ref[...] += st


def _pre1_body(x_ref, sa0, sa1, sb0, sb1, sd0, sd1, ws_ref, b_ref,
               h_ref, rdeg_ref, stats_ref):
    i = pl.program_id(0)
    ssum = jnp.concatenate([sa0[0] + sa1[0], sb0[0] + sb1[0]], axis=1)
    deg = (sd0[0] + sd1[0])[:, 0:1]
    rdeg = 1.0 / jnp.maximum(deg, 1.0)
    h = (jnp.dot(x_ref[...], ws_ref[...], preferred_element_type=jnp.float32)
         + ssum * rdeg + b_ref[...])
    h_ref[...] = h
    rdeg_ref[...] = rdeg
    _stats_update(i, h, stats_ref)


def _plane_specs(cols):
    return [pl.BlockSpec((1, BLK, cols), lambda i: (0, i, 0)),
            pl.BlockSpec((1, BLK, cols), lambda i: (1, i, 0))]


def _pre1(x_pad, Sa, Sb, Sd, Ws1, b1):
    return pl.pallas_call(
        _pre1_body,
        grid=(GRID,),
        in_specs=[pl.BlockSpec((BLK, D_IN), lambda i: (i, 0))]
        + _plane_specs(64) + _plane_specs(64) + _plane_specs(DDEG)
        + [
            pl.BlockSpec((D_IN, D_H), lambda i: (0, 0)),
            pl.BlockSpec((1, D_H), lambda i: (0, 0)),
        ],
        out_specs=[
            pl.BlockSpec((BLK, D_H), lambda i: (i, 0)),
            pl.BlockSpec((BLK, 1), lambda i: (i, 0)),
            pl.BlockSpec((2, D_H), lambda i: (0, 0)),
        ],
        out_shape=[
            jax.ShapeDtypeStruct((N_PAD, D_H), jnp.float32),
            jax.ShapeDtypeStruct((N_PAD, 1), jnp.float32),
            jax.ShapeDtypeStruct((2, D_H), jnp.float32),
        ],
    )(x_pad, Sa, Sa, Sb, Sb, Sd, Sd, Ws1, b1.reshape(1, -1))


def _pre2_body(hin_ref, sa0, sa1, sb0, sb1, rdeg_ref, ws_ref, b_ref,
               h_ref, stats_ref):
    i = pl.program_id(0)
    ssum = jnp.concatenate([sa0[0] + sa1[0], sb0[0] + sb1[0]], axis=1)
    h = (jnp.dot(hin_ref[...], ws_ref[...], preferred_element_type=jnp.float32)
         + ssum * rdeg_ref[...] + b_ref[...])
    h_ref[...] = h
    _stats_update(i, h, stats_ref)


def _pre2(h_in, Sa, Sb, rdeg, Ws, b):
    return pl.pallas_call(
        _pre2_body,
        grid=(GRID,),
        in_specs=[pl.BlockSpec((BLK, D_H), lambda i: (i, 0))]
        + _plane_specs(64) + _plane_specs(64)
        + [
            pl.BlockSpec((BLK, 1), lambda i: (i, 0)),
            pl.BlockSpec((D_H, D_H), lambda i: (0, 0)),
            pl.BlockSpec((1, D_H), lambda i: (0, 0)),
        ],
        out_specs=[
            pl.BlockSpec((BLK, D_H), lambda i: (i, 0)),
            pl.BlockSpec((2, D_H), lambda i: (0, 0)),
        ],
        out_shape=[
            jax.ShapeDtypeStruct((N_PAD, D_H), jnp.float32),
            jax.ShapeDtypeStruct((2, D_H), jnp.float32),
        ],
    )(h_in, Sa, Sa, Sb, Sb, rdeg, Ws, b.reshape(1, -1))


def _post_body(split, hpre_ref, stats_ref, g_ref, be_ref, wn_ref, hact_ref, *y_refs):
    mu = stats_ref[0:1, :] * (1.0 / N)
    var = stats_ref[1:2, :] * (1.0 / N) - mu * mu
    rstd = lax.rsqrt(var + 1e-5)
    h = (hpre_ref[...] - mu) * (rstd * g_ref[...]) + be_ref[...]
    h = jnp.maximum(h, 0.0)
    hact_ref[...] = h
    y = jnp.dot(h, wn_ref[...], preferred_element_type=jnp.float32)
    if split:
        y_refs[0][...] = y[:, :64]
        y_refs[1][...] = y[:, 64:]
    else:
        pad = y_refs[0].shape[1] - y.shape[1]
        if pad:
            y = jnp.concatenate([y, jnp.zeros((y.shape[0], pad), jnp.float32)], 1)
        y_refs[0][...] = y


def _post(hpre, stats, g, be, Wn_next, split, dn=None):
    if split:
        outs = [(N_PAD, D_H), (N_PAD, 64), (N_PAD, 64)]
        y_specs = [pl.BlockSpec((BLK, 64), lambda i: (i, 0)) for _ in range(2)]
    else:
        outs = [(N_PAD, D_H), (N_PAD, dn)]
        y_specs = [pl.BlockSpec((BLK, dn), lambda i: (i, 0))]
    return pl.pallas_call(
        functools.partial(_post_body, split),
        grid=(GRID,),
        in_specs=[
            pl.BlockSpec((BLK, D_H), lambda i: (i, 0)),
            pl.BlockSpec((2, D_H), lambda i: (0, 0)),
            pl.BlockSpec((1, D_H), lambda i: (0, 0)),
            pl.BlockSpec((1, D_H), lambda i: (0, 0)),
            pl.BlockSpec(Wn_next.shape, lambda i: (0, 0)),
        ],
        out_specs=[pl.BlockSpec((BLK, D_H), lambda i: (i, 0))] + y_specs,
        out_shape=[jax.ShapeDtypeStruct(o, jnp.float32) for o in outs],
    )(hpre, stats, g.reshape(1, -1), be.reshape(1, -1), Wn_next)


def _final_body(hin_ref, s0, s1, rdeg_ref, ws_ref, b_ref, out_ref):
    ssum = s0[0] + s1[0]
    h = (jnp.dot(hin_ref[...], ws_ref[...], preferred_element_type=jnp.float32)
         + ssum[:, :D_OUT] * rdeg_ref[...] + b_ref[...])
    m = jnp.max(h, axis=1, keepdims=True)
    e = jnp.exp(h - m)
    lse = jnp.log(jnp.sum(e, axis=1, keepdims=True))
    out_ref[...] = h - m - lse


def _final(h_in, S, rdeg, Ws3, b3):
    return pl.pallas_call(
        _final_body,
        grid=(GRID,),
        in_specs=[pl.BlockSpec((BLK, D_H), lambda i: (i, 0))]
        + _plane_specs(48)
        + [
            pl.BlockSpec((BLK, 1), lambda i: (i, 0)),
            pl.BlockSpec((D_H, D_OUT), lambda i: (0, 0)),
            pl.BlockSpec((1, D_OUT), lambda i: (0, 0)),
        ],
        out_specs=pl.BlockSpec((BLK, D_OUT), lambda i: (i, 0)),
        out_shape=jax.ShapeDtypeStruct((N_PAD, D_OUT), jnp.float32),
    )(h_in, S, S, rdeg, Ws3, b3.reshape(1, -1))


def kernel(x, edge_index, Ws1, Wn1, b1, g1, be1, Ws2, Wn2, b2, g2, be2, Ws3, Wn3, b3):
    x_pad = jnp.pad(x, ((0, N_PAD - N), (0, 0)))
    src = edge_index[0]
    dst = edge_index[1]
    src4 = jnp.pad(src, (0, E_PAD - E)).reshape(NTILE, NBLOCK, IB, CHUNK)
    dst4 = jnp.pad(dst, (0, E_PAD - E), constant_values=N).reshape(
        NTILE, NBLOCK, IB, CHUNK)

    y1a, y1b = _mm0(x_pad, Wn1)
    S1a, Sd = _segment_sum(y1a, src4, dst4, with_deg=True)
    S1b = _segment_sum(y1b, src4, dst4)
    hpre1, rdeg, st1 = _pre1(x_pad, S1a, S1b, Sd, Ws1, b1)
    h1, y2a, y2b = _post(hpre1, st1, g1, be1, Wn2, split=True)

    S2a = _segment_sum(y2a, src4, dst4)
    S2b = _segment_sum(y2b, src4, dst4)
    hpre2, st2 = _pre2(h1, S2a, S2b, rdeg, Ws2, b2)
    h2, y3 = _post(hpre2, st2, g2, be2, Wn3, split=False, dn=48)

    S3 = _segment_sum(y3, src4, dst4)
    out = _final(h2, S3, rdeg, Ws3, b3)
    return out[:N, :]
